# baseline (speedup 1.0000x reference)
import functools

import jax
import jax.numpy as jnp
from jax import lax
from jax.experimental import pallas as pl
from jax.experimental.pallas import tpu as pltpu
from jax.experimental.pallas import tpu_sc as plsc

NSUB = 16
NCORE = 2
NTILE = NSUB * NCORE
CH = 125


def _sc_mesh():
    return plsc.VectorSubcoreMesh(core_axis_name="c", subcore_axis_name="s")


def _sc_hist(dstp, zerosf):
    nt, ept = dstp.shape
    rows = 80
    d = 128

    @functools.partial(
        pl.kernel,
        out_type=jax.ShapeDtypeStruct((NTILE, rows * d), jnp.float32),
        mesh=_sc_mesh(),
        scratch_types=[
            pltpu.VMEM((ept,), jnp.int32),
            pltpu.VMEM((rows * d,), jnp.float32),
        ],
        compiler_params=pltpu.CompilerParams(needs_layout_passes=False),
    )
    def k(dst_h, z_h, out_h, didx, acc):
        cid = lax.axis_index("c")
        sid = lax.axis_index("s")
        tile = cid * NSUB + sid
        pltpu.sync_copy(z_h.at[tile], acc)
        pltpu.sync_copy(dst_h.at[tile], didx)

        def body(j, carry):
            idx = didx[pl.ds(j * 16, 16)]
            cnt, last = plsc.scan_count(idx)
            plsc.addupdate_scatter(
                acc, [idx], cnt.astype(jnp.float32), mask=last)
            return carry

        lax.fori_loop(0, ept // 16, body, 0)
        pltpu.sync_copy(acc, out_h.at[tile])

    return k(dstp, jnp.zeros((NTILE, rows * d), jnp.float32))


def _tc0(histp, ntile, rows):
    def body(h_ref, o_ref):
        h = h_ref[...].reshape(ntile, rows, h_ref.shape[-1])
        o_ref[...] = jnp.sum(h, axis=0)

    return pl.pallas_call(
        body,
        out_shape=jax.ShapeDtypeStruct((rows, histp.shape[-1]), jnp.float32),
    )(histp)


def _sc_scatter(g, src2d, dst2d, zerosf):
    d = g.shape[1]
    n = zerosf.shape[0]
    r, ch = src2d.shape
    rpt = r // NTILE
    rz = n // NSUB
    br = 40
    nblk = rpt // br

    @functools.partial(
        pl.kernel,
        out_type=jax.ShapeDtypeStruct((NCORE * n, d), jnp.float32),
        mesh=_sc_mesh(),
        scratch_types=[
            pltpu.VMEM((br, ch), jnp.int32),
            pltpu.VMEM((br, ch), jnp.int32),
            pltpu.VMEM((ch, d), jnp.float32),
            pltpu.VMEM((ch, d), jnp.float32),
            pltpu.VMEM_SHARED((n, d), jnp.float32),
            pltpu.SemaphoreType.DMA,
            pltpu.SemaphoreType.DMA,
            pltpu.SemaphoreType.DMA,
            pltpu.SemaphoreType.DMA,
        ],
    )
    def k(g_h, s_h, d_h, z_h, out_h, sidx, didx, rows0, rows1, acc,
          sem0, sem1, ssem0, ssem1):
        cid = lax.axis_index("c")
        sid = lax.axis_index("s")
        tile = cid * NSUB + sid
        pltpu.sync_copy(z_h.at[pl.ds(sid * rz, rz)], acc.at[pl.ds(sid * rz, rz)])
        plsc.subcore_barrier()

        def blk(bi, carry):
            base = tile * rpt + bi * br
            pltpu.sync_copy(s_h.at[pl.ds(base, br)], sidx)
            pltpu.sync_copy(d_h.at[pl.ds(base, br)], didx)

            pltpu.async_copy(g_h.at[sidx.at[0]], rows0, sem0)
            pltpu.async_copy(g_h.at[sidx.at[1]], rows1, sem1)

            def body(p, c):
                j0 = 2 * p
                j1 = j0 + 1
                pltpu.make_async_copy(g_h.at[sidx.at[j0]], rows0, sem0).wait()
                pltpu.async_copy(rows0, acc.at[didx.at[j0]], ssem0, add=True)
                pltpu.make_async_copy(g_h.at[sidx.at[j1]], rows1, sem1).wait()
                pltpu.async_copy(rows1, acc.at[didx.at[j1]], ssem1, add=True)

                @pl.when(j0 + 2 < br)
                def _():
                    pltpu.make_async_copy(
                        rows0, acc.at[didx.at[j0]], ssem0).wait()
                    pltpu.async_copy(g_h.at[sidx.at[j0 + 2]], rows0, sem0)

                @pl.when(j1 + 2 < br)
                def _():
                    pltpu.make_async_copy(
                        rows1, acc.at[didx.at[j1]], ssem1).wait()
                    pltpu.async_copy(g_h.at[sidx.at[j1 + 2]], rows1, sem1)

                return c

            lax.fori_loop(0, br // 2, body, 0)
            pltpu.make_async_copy(rows0, acc.at[didx.at[br - 2]], ssem0).wait()
            pltpu.make_async_copy(rows1, acc.at[didx.at[br - 1]], ssem1).wait()
            return carry

        lax.fori_loop(0, nblk, blk, 0)
        plsc.subcore_barrier()
        pltpu.sync_copy(acc.at[pl.ds(sid * rz, rz)],
                        out_h.at[pl.ds(cid * n + sid * rz, rz)])

    return k(g, src2d, dst2d, zerosf)


def _tc1(x, w0, degp, nb=10):
    n, din = x.shape
    dh = w0.shape[1]
    br = n // nb

    def body(x_ref, w_ref, deg_ref, g0_ref, sl0_ref, sc2_ref):
        deg = 1.0 + deg_ref[...]
        dis = lax.rsqrt(deg)
        dis2 = 1.0 / deg
        h = jnp.dot(x_ref[...], w_ref[...], preferred_element_type=jnp.float32)
        g0_ref[...] = h * dis
        sl0_ref[...] = h * dis2
        sc2_ref[...] = jnp.concatenate([dis, dis2], axis=1)

    return pl.pallas_call(
        body,
        grid=(nb,),
        in_specs=[
            pl.BlockSpec((br, din), lambda i: (i, 0)),
            pl.BlockSpec((din, dh), lambda i: (0, 0)),
            pl.BlockSpec((br, 1), lambda i: (i, 0)),
        ],
        out_specs=[
            pl.BlockSpec((br, dh), lambda i: (i, 0)),
            pl.BlockSpec((br, dh), lambda i: (i, 0)),
            pl.BlockSpec((br, 2), lambda i: (i, 0)),
        ],
        out_shape=[
            jax.ShapeDtypeStruct((n, dh), jnp.float32),
            jax.ShapeDtypeStruct((n, dh), jnp.float32),
            jax.ShapeDtypeStruct((n, 2), jnp.float32),
        ],
    )(x, w0, degp)


def _tc2(p0, p1, sl0, sc2, b0, w1, nb=10):
    n, dh = sl0.shape
    d2 = w1.shape[1]
    br = n // nb

    def body(p0_ref, p1_ref, sl_ref, sc_ref, b_ref, w_ref, g_ref, sl1_ref):
        dis = sc_ref[...][:, 0:1]
        dis2 = sc_ref[...][:, 1:2]
        x1 = jax.nn.relu(dis * (p0_ref[...] + p1_ref[...]) + sl_ref[...]
                         + b_ref[...])
        h = jnp.dot(x1, w_ref[...], preferred_element_type=jnp.float32)
        g_ref[...] = h * dis
        sl1_ref[...] = h * dis2

    return pl.pallas_call(
        body,
        grid=(nb,),
        in_specs=[
            pl.BlockSpec((br, dh), lambda i: (i, 0)),
            pl.BlockSpec((br, dh), lambda i: (i, 0)),
            pl.BlockSpec((br, dh), lambda i: (i, 0)),
            pl.BlockSpec((br, 2), lambda i: (i, 0)),
            pl.BlockSpec((1, dh), lambda i: (0, 0)),
            pl.BlockSpec((dh, d2), lambda i: (0, 0)),
        ],
        out_specs=[
            pl.BlockSpec((br, d2), lambda i: (i, 0)),
            pl.BlockSpec((br, d2), lambda i: (i, 0)),
        ],
        out_shape=[
            jax.ShapeDtypeStruct((n, d2), jnp.float32),
            jax.ShapeDtypeStruct((n, d2), jnp.float32),
        ],
    )(p0, p1, sl0, sc2, b0, w1)


def _tc3(p0, p1, sl1, sc2, b1, wfc, bfc, nb=10):
    n, dh = sl1.shape
    dout = wfc.shape[1]
    br = n // nb

    def body(p0_ref, p1_ref, sl_ref, sc_ref, b_ref, w_ref, bf_ref,
             x2_ref, pr_ref):
        dis = sc_ref[...][:, 0:1]
        x2 = jax.nn.relu(dis * (p0_ref[...] + p1_ref[...]) + sl_ref[...]
                         + b_ref[...])
        x2_ref[...] = x2
        z = jnp.dot(x2, w_ref[...], preferred_element_type=jnp.float32)
        pr_ref[...] = jax.nn.sigmoid(z + bf_ref[...])

    return pl.pallas_call(
        body,
        grid=(nb,),
        in_specs=[
            pl.BlockSpec((br, dh), lambda i: (i, 0)),
            pl.BlockSpec((br, dh), lambda i: (i, 0)),
            pl.BlockSpec((br, dh), lambda i: (i, 0)),
            pl.BlockSpec((br, 2), lambda i: (i, 0)),
            pl.BlockSpec((1, dh), lambda i: (0, 0)),
            pl.BlockSpec((dh, dout), lambda i: (0, 0)),
            pl.BlockSpec((1, dout), lambda i: (0, 0)),
        ],
        out_specs=[
            pl.BlockSpec((br, dh), lambda i: (i, 0)),
            pl.BlockSpec((br, dout), lambda i: (i, 0)),
        ],
        out_shape=[
            jax.ShapeDtypeStruct((n, dh), jnp.float32),
            jax.ShapeDtypeStruct((n, dout), jnp.float32),
        ],
    )(p0, p1, sl1, sc2, b1, wfc, bfc)


def _tc4(praw):
    def body(p_ref, o_ref):
        p = p_ref[...]
        lo = jnp.min(p)
        hi = jnp.max(p)
        o_ref[...] = (p - lo) / (hi - lo)

    return pl.pallas_call(
        body,
        out_shape=jax.ShapeDtypeStruct(praw.shape, jnp.float32),
    )(praw)


def kernel(x, edge_index, W0, b0, W1, b1, Wfc, bfc):
    n, din = x.shape
    e = edge_index.shape[1]
    ei = edge_index.astype(jnp.int32)
    r = e // CH
    src2d = ei[0].reshape(r, CH)
    dst2d = ei[1].reshape(r, CH)

    npad = ((n + 127) // 128) * 128
    zerosf = jnp.zeros((npad, W0.shape[1]), jnp.float32)

    hrows = 80
    eg = NTILE * 128 * 8
    epad = ((e + eg - 1) // eg) * eg
    dstp = jnp.concatenate(
        [ei[1], jnp.full((epad - e,), npad - 1, jnp.int32)]).reshape(NTILE, -1)

    histp = _sc_hist(dstp, zerosf)
    deg = _tc0(histp.reshape(NTILE * hrows, 128), NTILE, hrows)
    degp = deg.reshape(-1)[:n].reshape(n, 1)
    g0, sl0, sc2 = _tc1(x, W0, degp)
    parts0 = _sc_scatter(g0, src2d, dst2d, zerosf)
    g1, sl1 = _tc2(parts0[:n], parts0[npad:npad + n], sl0, sc2,
                   b0.reshape(1, -1), W1)
    parts1 = _sc_scatter(g1, src2d, dst2d, zerosf)
    x_pre_fc, praw = _tc3(parts1[:n], parts1[npad:npad + n], sl1, sc2,
                          b1.reshape(1, -1), Wfc, bfc.reshape(1, -1))
    x_post_fc = _tc4(praw)
    return (x_pre_fc, x_post_fc)


# speedup vs baseline: 1.2146x; 1.2146x over previous
import functools

import jax
import jax.numpy as jnp
from jax import lax
from jax.experimental import pallas as pl
from jax.experimental.pallas import tpu as pltpu
from jax.experimental.pallas import tpu_sc as plsc

NSUB = 16
NCORE = 2
NTILE = NSUB * NCORE
CH = 125


def _sc_mesh():
    return plsc.VectorSubcoreMesh(core_axis_name="c", subcore_axis_name="s")


def _sc_hist(dstp, zerosf):
    nt, ept = dstp.shape
    rows = 80
    d = 128

    @functools.partial(
        pl.kernel,
        out_type=jax.ShapeDtypeStruct((NTILE, rows * d), jnp.float32),
        mesh=_sc_mesh(),
        scratch_types=[
            pltpu.VMEM((ept,), jnp.int32),
            pltpu.VMEM((rows * d,), jnp.float32),
        ],
        compiler_params=pltpu.CompilerParams(needs_layout_passes=False),
    )
    def k(dst_h, z_h, out_h, didx, acc):
        cid = lax.axis_index("c")
        sid = lax.axis_index("s")
        tile = cid * NSUB + sid
        pltpu.sync_copy(z_h.at[tile], acc)
        pltpu.sync_copy(dst_h.at[tile], didx)

        def body(j, carry):
            idx = didx[pl.ds(j * 16, 16)]
            cnt, last = plsc.scan_count(idx)
            plsc.addupdate_scatter(
                acc, [idx], cnt.astype(jnp.float32), mask=last)
            return carry

        lax.fori_loop(0, ept // 16, body, 0)
        pltpu.sync_copy(acc, out_h.at[tile])

    return k(dstp, jnp.zeros((NTILE, rows * d), jnp.float32))


def _tc0(histp, ntile, rows):
    def body(h_ref, o_ref):
        h = h_ref[...].reshape(ntile, rows, h_ref.shape[-1])
        o_ref[...] = jnp.sum(h, axis=0)

    return pl.pallas_call(
        body,
        out_shape=jax.ShapeDtypeStruct((rows, histp.shape[-1]), jnp.float32),
    )(histp)


def _sc_scatter(g, src2d, dst2d, zerosf):
    d = g.shape[1]
    n = zerosf.shape[0]
    r, ch = src2d.shape
    rpt = r // NTILE
    rz = n // NSUB
    br = 40
    nblk = rpt // br

    @functools.partial(
        pl.kernel,
        out_type=jax.ShapeDtypeStruct((NCORE * n, d), jnp.float32),
        mesh=_sc_mesh(),
        scratch_types=[
            pltpu.VMEM((br, ch), jnp.int32),
            pltpu.VMEM((br, ch), jnp.int32),
            pltpu.VMEM((ch, d), jnp.float32),
            pltpu.VMEM((ch, d), jnp.float32),
            pltpu.VMEM_SHARED((n, d), jnp.float32),
            pltpu.SemaphoreType.DMA,
            pltpu.SemaphoreType.DMA,
        ],
    )
    def k(g_h, s_h, d_h, z_h, out_h, sidx, didx, rows0, rows1, acc,
          sem0, sem1):
        cid = lax.axis_index("c")
        sid = lax.axis_index("s")
        tile = cid * NSUB + sid
        pltpu.sync_copy(z_h.at[pl.ds(sid * rz, rz)], acc.at[pl.ds(sid * rz, rz)])
        plsc.subcore_barrier()

        def blk(bi, carry):
            base = tile * rpt + bi * br
            pltpu.sync_copy(s_h.at[pl.ds(base, br)], sidx)
            pltpu.sync_copy(d_h.at[pl.ds(base, br)], didx)

            pltpu.async_copy(g_h.at[sidx.at[0]], rows0, sem0)
            pltpu.async_copy(g_h.at[sidx.at[1]], rows1, sem1)

            def body(p, c):
                j0 = 2 * p
                j1 = j0 + 1
                pltpu.make_async_copy(g_h.at[sidx.at[j0]], rows0, sem0).wait()
                pltpu.sync_copy(rows0, acc.at[didx.at[j0]], add=True)

                @pl.when(j0 + 2 < br)
                def _():
                    pltpu.async_copy(g_h.at[sidx.at[j0 + 2]], rows0, sem0)

                pltpu.make_async_copy(g_h.at[sidx.at[j1]], rows1, sem1).wait()
                pltpu.sync_copy(rows1, acc.at[didx.at[j1]], add=True)

                @pl.when(j1 + 2 < br)
                def _():
                    pltpu.async_copy(g_h.at[sidx.at[j1 + 2]], rows1, sem1)

                return c

            lax.fori_loop(0, br // 2, body, 0)
            return carry

        lax.fori_loop(0, nblk, blk, 0)
        plsc.subcore_barrier()
        pltpu.sync_copy(acc.at[pl.ds(sid * rz, rz)],
                        out_h.at[pl.ds(cid * n + sid * rz, rz)])

    return k(g, src2d, dst2d, zerosf)


def _tc1(x, w0, degp, nb=10):
    n, din = x.shape
    dh = w0.shape[1]
    br = n // nb

    def body(x_ref, w_ref, deg_ref, g0_ref, sl0_ref, sc2_ref):
        deg = 1.0 + deg_ref[...]
        dis = lax.rsqrt(deg)
        dis2 = 1.0 / deg
        h = jnp.dot(x_ref[...], w_ref[...], preferred_element_type=jnp.float32)
        g0_ref[...] = h * dis
        sl0_ref[...] = h * dis2
        sc2_ref[...] = jnp.concatenate([dis, dis2], axis=1)

    return pl.pallas_call(
        body,
        grid=(nb,),
        in_specs=[
            pl.BlockSpec((br, din), lambda i: (i, 0)),
            pl.BlockSpec((din, dh), lambda i: (0, 0)),
            pl.BlockSpec((br, 1), lambda i: (i, 0)),
        ],
        out_specs=[
            pl.BlockSpec((br, dh), lambda i: (i, 0)),
            pl.BlockSpec((br, dh), lambda i: (i, 0)),
            pl.BlockSpec((br, 2), lambda i: (i, 0)),
        ],
        out_shape=[
            jax.ShapeDtypeStruct((n, dh), jnp.float32),
            jax.ShapeDtypeStruct((n, dh), jnp.float32),
            jax.ShapeDtypeStruct((n, 2), jnp.float32),
        ],
    )(x, w0, degp)


def _tc2(p0, p1, sl0, sc2, b0, w1, nb=10):
    n, dh = sl0.shape
    d2 = w1.shape[1]
    br = n // nb

    def body(p0_ref, p1_ref, sl_ref, sc_ref, b_ref, w_ref, g_ref, sl1_ref):
        dis = sc_ref[...][:, 0:1]
        dis2 = sc_ref[...][:, 1:2]
        x1 = jax.nn.relu(dis * (p0_ref[...] + p1_ref[...]) + sl_ref[...]
                         + b_ref[...])
        h = jnp.dot(x1, w_ref[...], preferred_element_type=jnp.float32)
        g_ref[...] = h * dis
        sl1_ref[...] = h * dis2

    return pl.pallas_call(
        body,
        grid=(nb,),
        in_specs=[
            pl.BlockSpec((br, dh), lambda i: (i, 0)),
            pl.BlockSpec((br, dh), lambda i: (i, 0)),
            pl.BlockSpec((br, dh), lambda i: (i, 0)),
            pl.BlockSpec((br, 2), lambda i: (i, 0)),
            pl.BlockSpec((1, dh), lambda i: (0, 0)),
            pl.BlockSpec((dh, d2), lambda i: (0, 0)),
        ],
        out_specs=[
            pl.BlockSpec((br, d2), lambda i: (i, 0)),
            pl.BlockSpec((br, d2), lambda i: (i, 0)),
        ],
        out_shape=[
            jax.ShapeDtypeStruct((n, d2), jnp.float32),
            jax.ShapeDtypeStruct((n, d2), jnp.float32),
        ],
    )(p0, p1, sl0, sc2, b0, w1)


def _tc3(p0, p1, sl1, sc2, b1, wfc, bfc, nb=10):
    n, dh = sl1.shape
    dout = wfc.shape[1]
    br = n // nb

    def body(p0_ref, p1_ref, sl_ref, sc_ref, b_ref, w_ref, bf_ref,
             x2_ref, o2_ref, pr_s, mm_s):
        i = pl.program_id(0)

        @pl.when(i < nb)
        def _():
            dis = sc_ref[...][:, 0:1]
            x2 = jax.nn.relu(dis * (p0_ref[...] + p1_ref[...]) + sl_ref[...]
                             + b_ref[...])
            x2_ref[...] = x2
            z = jnp.dot(x2, w_ref[...], preferred_element_type=jnp.float32)
            praw = jax.nn.sigmoid(z + bf_ref[...])
            pr_s[pl.ds(i * br, br), :] = praw
            blo = jnp.min(praw)
            bhi = jnp.max(praw)

            @pl.when(i == 0)
            def _():
                mm_s[0, 0] = blo
                mm_s[0, 1] = bhi

            @pl.when(i > 0)
            def _():
                mm_s[0, 0] = jnp.minimum(mm_s[0, 0], blo)
                mm_s[0, 1] = jnp.maximum(mm_s[0, 1], bhi)

        @pl.when(i == nb)
        def _():
            lo = mm_s[0, 0]
            hi = mm_s[0, 1]
            o2_ref[...] = (pr_s[...] - lo) / (hi - lo)

    return pl.pallas_call(
        body,
        grid=(nb + 1,),
        in_specs=[
            pl.BlockSpec((br, dh), lambda i: (jnp.minimum(i, nb - 1), 0)),
            pl.BlockSpec((br, dh), lambda i: (jnp.minimum(i, nb - 1), 0)),
            pl.BlockSpec((br, dh), lambda i: (jnp.minimum(i, nb - 1), 0)),
            pl.BlockSpec((br, 2), lambda i: (jnp.minimum(i, nb - 1), 0)),
            pl.BlockSpec((1, dh), lambda i: (0, 0)),
            pl.BlockSpec((dh, dout), lambda i: (0, 0)),
            pl.BlockSpec((1, dout), lambda i: (0, 0)),
        ],
        out_specs=[
            pl.BlockSpec((br, dh), lambda i: (jnp.minimum(i, nb - 1), 0)),
            pl.BlockSpec((n, dout), lambda i: (0, 0)),
        ],
        out_shape=[
            jax.ShapeDtypeStruct((n, dh), jnp.float32),
            jax.ShapeDtypeStruct((n, dout), jnp.float32),
        ],
        scratch_shapes=[
            pltpu.VMEM((n, dout), jnp.float32),
            pltpu.SMEM((1, 2), jnp.float32),
        ],
    )(p0, p1, sl1, sc2, b1, wfc, bfc)


def kernel(x, edge_index, W0, b0, W1, b1, Wfc, bfc):
    n, din = x.shape
    e = edge_index.shape[1]
    ei = edge_index.astype(jnp.int32)
    r = e // CH
    src2d = ei[0].reshape(r, CH)
    dst2d = ei[1].reshape(r, CH)

    npad = ((n + 127) // 128) * 128
    zerosf = jnp.zeros((npad, W0.shape[1]), jnp.float32)

    hrows = 80
    eg = NTILE * 128 * 8
    epad = ((e + eg - 1) // eg) * eg
    dstp = jnp.concatenate(
        [ei[1], jnp.full((epad - e,), npad - 1, jnp.int32)]).reshape(NTILE, -1)

    histp = _sc_hist(dstp, zerosf)
    deg = _tc0(histp.reshape(NTILE * hrows, 128), NTILE, hrows)
    degp = deg.reshape(-1)[:n].reshape(n, 1)
    g0, sl0, sc2 = _tc1(x, W0, degp)
    parts0 = _sc_scatter(g0, src2d, dst2d, zerosf)
    g1, sl1 = _tc2(parts0[:n], parts0[npad:npad + n], sl0, sc2,
                   b0.reshape(1, -1), W1)
    parts1 = _sc_scatter(g1, src2d, dst2d, zerosf)
    x_pre_fc, x_post_fc = _tc3(parts1[:n], parts1[npad:npad + n], sl1, sc2,
                               b1.reshape(1, -1), Wfc, bfc.reshape(1, -1))
    return (x_pre_fc, x_post_fc)


# speedup vs baseline: 1.2350x; 1.0168x over previous
import functools

import jax
import jax.numpy as jnp
from jax import lax
from jax.experimental import pallas as pl
from jax.experimental.pallas import tpu as pltpu
from jax.experimental.pallas import tpu_sc as plsc

NSUB = 16
NCORE = 2
NTILE = NSUB * NCORE
CH = 125


def _sc_mesh():
    return plsc.VectorSubcoreMesh(core_axis_name="c", subcore_axis_name="s")


def _sc_hist(dstp, zerosf):
    nt, ept = dstp.shape
    rows = 80
    d = 128

    @functools.partial(
        pl.kernel,
        out_type=jax.ShapeDtypeStruct((NTILE, rows * d), jnp.float32),
        mesh=_sc_mesh(),
        scratch_types=[
            pltpu.VMEM((ept,), jnp.int32),
            pltpu.VMEM((rows * d,), jnp.float32),
        ],
        compiler_params=pltpu.CompilerParams(needs_layout_passes=False),
    )
    def k(dst_h, z_h, out_h, didx, acc):
        cid = lax.axis_index("c")
        sid = lax.axis_index("s")
        tile = cid * NSUB + sid
        pltpu.sync_copy(z_h.at[tile], acc)
        pltpu.sync_copy(dst_h.at[tile], didx)

        def body(j, carry):
            idx = didx[pl.ds(j * 16, 16)]
            cnt, last = plsc.scan_count(idx)
            plsc.addupdate_scatter(
                acc, [idx], cnt.astype(jnp.float32), mask=last)
            return carry

        lax.fori_loop(0, ept // 16, body, 0)
        pltpu.sync_copy(acc, out_h.at[tile])

    return k(dstp, jnp.zeros((NTILE, rows * d), jnp.float32))


def _tc0(histp, ntile, rows):
    def body(h_ref, o_ref):
        h = h_ref[...].reshape(ntile, rows, h_ref.shape[-1])
        o_ref[...] = jnp.sum(h, axis=0)

    return pl.pallas_call(
        body,
        out_shape=jax.ShapeDtypeStruct((rows, histp.shape[-1]), jnp.float32),
    )(histp)


def _sc_scatter(g, src2d, dst2d, zerosf):
    d = g.shape[1]
    n = zerosf.shape[0]
    r, ch = src2d.shape
    rpt = r // NTILE
    rz = n // NSUB
    br = 40
    nblk = rpt // br

    @functools.partial(
        pl.kernel,
        out_type=jax.ShapeDtypeStruct((NCORE * n, d), jnp.float32),
        mesh=_sc_mesh(),
        scratch_types=[
            pltpu.VMEM((br, ch), jnp.int32),
            pltpu.VMEM((br, ch), jnp.int32),
            pltpu.VMEM((ch, d), jnp.float32),
            pltpu.VMEM((ch, d), jnp.float32),
            pltpu.VMEM_SHARED((n, d), jnp.float32),
            pltpu.SemaphoreType.DMA,
            pltpu.SemaphoreType.DMA,
        ],
    )
    def k(g_h, s_h, d_h, z_h, out_h, sidx, didx, rows0, rows1, acc,
          sem0, sem1):
        cid = lax.axis_index("c")
        sid = lax.axis_index("s")
        tile = cid * NSUB + sid
        pltpu.sync_copy(z_h.at[pl.ds(sid * rz, rz)], acc.at[pl.ds(sid * rz, rz)])
        plsc.subcore_barrier()

        def blk(bi, carry):
            base = tile * rpt + bi * br
            pltpu.sync_copy(s_h.at[pl.ds(base, br)], sidx)
            pltpu.sync_copy(d_h.at[pl.ds(base, br)], didx)

            pltpu.async_copy(g_h.at[sidx.at[0]], rows0, sem0)
            pltpu.async_copy(g_h.at[sidx.at[1]], rows1, sem1)

            def body(p, c):
                j0 = 2 * p
                j1 = j0 + 1
                pltpu.make_async_copy(g_h.at[sidx.at[j0]], rows0, sem0).wait()
                pltpu.sync_copy(rows0, acc.at[didx.at[j0]], add=True)

                @pl.when(j0 + 2 < br)
                def _():
                    pltpu.async_copy(g_h.at[sidx.at[j0 + 2]], rows0, sem0)

                pltpu.make_async_copy(g_h.at[sidx.at[j1]], rows1, sem1).wait()
                pltpu.sync_copy(rows1, acc.at[didx.at[j1]], add=True)

                @pl.when(j1 + 2 < br)
                def _():
                    pltpu.async_copy(g_h.at[sidx.at[j1 + 2]], rows1, sem1)

                return c

            lax.fori_loop(0, br // 2, body, 0)
            return carry

        lax.fori_loop(0, nblk, blk, 0)
        plsc.subcore_barrier()
        pltpu.sync_copy(acc.at[pl.ds(sid * rz, rz)],
                        out_h.at[pl.ds(cid * n + sid * rz, rz)])

    return k(g, src2d, dst2d, zerosf)


def _tc1(x, w0, degp, nb=5):
    n, din = x.shape
    dh = w0.shape[1]
    br = n // nb

    def body(x_ref, w_ref, deg_ref, g0_ref, sl0_ref, sc2_ref):
        deg = 1.0 + deg_ref[...]
        dis = lax.rsqrt(deg)
        dis2 = 1.0 / deg
        h = jnp.dot(x_ref[...], w_ref[...], preferred_element_type=jnp.float32)
        g0_ref[...] = h * dis
        sl0_ref[...] = h * dis2
        sc2_ref[...] = jnp.concatenate([dis, dis2], axis=1)

    return pl.pallas_call(
        body,
        grid=(nb,),
        in_specs=[
            pl.BlockSpec((br, din), lambda i: (i, 0)),
            pl.BlockSpec((din, dh), lambda i: (0, 0)),
            pl.BlockSpec((br, 1), lambda i: (i, 0)),
        ],
        out_specs=[
            pl.BlockSpec((br, dh), lambda i: (i, 0)),
            pl.BlockSpec((br, dh), lambda i: (i, 0)),
            pl.BlockSpec((br, 2), lambda i: (i, 0)),
        ],
        out_shape=[
            jax.ShapeDtypeStruct((n, dh), jnp.float32),
            jax.ShapeDtypeStruct((n, dh), jnp.float32),
            jax.ShapeDtypeStruct((n, 2), jnp.float32),
        ],
    )(x, w0, degp)


def _tc2(p0, p1, sl0, sc2, b0, w1, nb=5):
    n, dh = sl0.shape
    d2 = w1.shape[1]
    br = n // nb

    def body(p0_ref, p1_ref, sl_ref, sc_ref, b_ref, w_ref, g_ref, sl1_ref):
        dis = sc_ref[...][:, 0:1]
        dis2 = sc_ref[...][:, 1:2]
        x1 = jax.nn.relu(dis * (p0_ref[...] + p1_ref[...]) + sl_ref[...]
                         + b_ref[...])
        h = jnp.dot(x1, w_ref[...], preferred_element_type=jnp.float32)
        g_ref[...] = h * dis
        sl1_ref[...] = h * dis2

    return pl.pallas_call(
        body,
        grid=(nb,),
        in_specs=[
            pl.BlockSpec((br, dh), lambda i: (i, 0)),
            pl.BlockSpec((br, dh), lambda i: (i, 0)),
            pl.BlockSpec((br, dh), lambda i: (i, 0)),
            pl.BlockSpec((br, 2), lambda i: (i, 0)),
            pl.BlockSpec((1, dh), lambda i: (0, 0)),
            pl.BlockSpec((dh, d2), lambda i: (0, 0)),
        ],
        out_specs=[
            pl.BlockSpec((br, d2), lambda i: (i, 0)),
            pl.BlockSpec((br, d2), lambda i: (i, 0)),
        ],
        out_shape=[
            jax.ShapeDtypeStruct((n, d2), jnp.float32),
            jax.ShapeDtypeStruct((n, d2), jnp.float32),
        ],
    )(p0, p1, sl0, sc2, b0, w1)


def _tc3(p0, p1, sl1, sc2, b1, wfc, bfc, nb=5):
    n, dh = sl1.shape
    dout = wfc.shape[1]
    br = n // nb

    def body(p0_ref, p1_ref, sl_ref, sc_ref, b_ref, w_ref, bf_ref,
             x2_ref, o2_ref, pr_s, mm_s):
        i = pl.program_id(0)

        @pl.when(i < nb)
        def _():
            dis = sc_ref[...][:, 0:1]
            x2 = jax.nn.relu(dis * (p0_ref[...] + p1_ref[...]) + sl_ref[...]
                             + b_ref[...])
            x2_ref[...] = x2
            z = jnp.dot(x2, w_ref[...], preferred_element_type=jnp.float32)
            praw = jax.nn.sigmoid(z + bf_ref[...])
            pr_s[pl.ds(i * br, br), :] = praw
            blo = jnp.min(praw)
            bhi = jnp.max(praw)

            @pl.when(i == 0)
            def _():
                mm_s[0, 0] = blo
                mm_s[0, 1] = bhi

            @pl.when(i > 0)
            def _():
                mm_s[0, 0] = jnp.minimum(mm_s[0, 0], blo)
                mm_s[0, 1] = jnp.maximum(mm_s[0, 1], bhi)

        @pl.when(i == nb)
        def _():
            lo = mm_s[0, 0]
            hi = mm_s[0, 1]
            o2_ref[...] = (pr_s[...] - lo) / (hi - lo)

    return pl.pallas_call(
        body,
        grid=(nb + 1,),
        in_specs=[
            pl.BlockSpec((br, dh), lambda i: (jnp.minimum(i, nb - 1), 0)),
            pl.BlockSpec((br, dh), lambda i: (jnp.minimum(i, nb - 1), 0)),
            pl.BlockSpec((br, dh), lambda i: (jnp.minimum(i, nb - 1), 0)),
            pl.BlockSpec((br, 2), lambda i: (jnp.minimum(i, nb - 1), 0)),
            pl.BlockSpec((1, dh), lambda i: (0, 0)),
            pl.BlockSpec((dh, dout), lambda i: (0, 0)),
            pl.BlockSpec((1, dout), lambda i: (0, 0)),
        ],
        out_specs=[
            pl.BlockSpec((br, dh), lambda i: (jnp.minimum(i, nb - 1), 0)),
            pl.BlockSpec((n, dout), lambda i: (0, 0)),
        ],
        out_shape=[
            jax.ShapeDtypeStruct((n, dh), jnp.float32),
            jax.ShapeDtypeStruct((n, dout), jnp.float32),
        ],
        scratch_shapes=[
            pltpu.VMEM((n, dout), jnp.float32),
            pltpu.SMEM((1, 2), jnp.float32),
        ],
    )(p0, p1, sl1, sc2, b1, wfc, bfc)


def kernel(x, edge_index, W0, b0, W1, b1, Wfc, bfc):
    n, din = x.shape
    e = edge_index.shape[1]
    ei = edge_index.astype(jnp.int32)
    r = e // CH
    src2d = ei[0].reshape(r, CH)
    dst2d = ei[1].reshape(r, CH)

    npad = ((n + 127) // 128) * 128
    zerosf = jnp.zeros((npad, W0.shape[1]), jnp.float32)

    hrows = 80
    eg = NTILE * 128 * 8
    epad = ((e + eg - 1) // eg) * eg
    dstp = jnp.concatenate(
        [ei[1], jnp.full((epad - e,), npad - 1, jnp.int32)]).reshape(NTILE, -1)

    histp = _sc_hist(dstp, zerosf)
    deg = _tc0(histp.reshape(NTILE * hrows, 128), NTILE, hrows)
    degp = deg.reshape(-1)[:n].reshape(n, 1)
    g0, sl0, sc2 = _tc1(x, W0, degp)
    parts0 = _sc_scatter(g0, src2d, dst2d, zerosf)
    g1, sl1 = _tc2(parts0[:n], parts0[npad:npad + n], sl0, sc2,
                   b0.reshape(1, -1), W1)
    parts1 = _sc_scatter(g1, src2d, dst2d, zerosf)
    x_pre_fc, x_post_fc = _tc3(parts1[:n], parts1[npad:npad + n], sl1, sc2,
                               b1.reshape(1, -1), Wfc, bfc.reshape(1, -1))
    return (x_pre_fc, x_post_fc)
